# Initial kernel scaffold; baseline (speedup 1.0000x reference)
import jax
import jax.numpy as jnp
from jax.experimental import pallas as pl


def kernel(pointcloud, params):
    raise NotImplementedError("write your pallas kernel here")


# speedup vs baseline: 17.7506x; 17.7506x over previous
import functools

import jax
import jax.numpy as jnp
from jax import lax
from jax.experimental import pallas as pl
from jax.experimental.pallas import tpu as pltpu
from jax.experimental.pallas import tpu_sc as plsc
import numpy as np

_BN_SCALE = 1.0 / np.sqrt(1.0 + 1e-5)
_SA_CFG = [(2048, 0.1, 32), (512, 0.2, 32), (128, 0.4, 32), (32, 0.8, 32)]
_B = 4
_N = 8192
_SC_NC = 2
_SC_NS = 16
_SC_NW = _SC_NC * _SC_NS


def _fps_body(x_ref, y_ref, z_ref, *out_refs):
    X = x_ref[...]
    Y = y_ref[...]
    Z = z_ref[...]
    npoints = [c[0] for c in _SA_CFG]
    for lvl, M in enumerate(npoints):
        Nl = X.shape[1]
        iota_n = lax.broadcasted_iota(jnp.int32, (_B, Nl), 1).astype(jnp.float32)
        iota_m = lax.broadcasted_iota(jnp.int32, (_B, M), 1).astype(jnp.float32)

        def step(s, carry, X=X, Y=Y, Z=Z, iota_n=iota_n, iota_m=iota_m, Nl=Nl):
            dists, cx, cy, cz, ax, ay, az = carry
            sel = iota_m == s.astype(jnp.float32)
            ax = jnp.where(sel, cx, ax)
            ay = jnp.where(sel, cy, ay)
            az = jnp.where(sel, cz, az)
            d = (X - cx) ** 2 + (Y - cy) ** 2 + (Z - cz) ** 2
            dists = jnp.minimum(dists, d)
            m = jnp.max(dists, axis=1, keepdims=True)
            far = jnp.min(jnp.where(dists == m, iota_n, float(Nl)),
                          axis=1, keepdims=True)
            fmask = iota_n == far
            cx = jnp.sum(jnp.where(fmask, X, 0.0), axis=1, keepdims=True)
            cy = jnp.sum(jnp.where(fmask, Y, 0.0), axis=1, keepdims=True)
            cz = jnp.sum(jnp.where(fmask, Z, 0.0), axis=1, keepdims=True)
            return (dists, cx, cy, cz, ax, ay, az)

        init = (jnp.full((_B, Nl), 1e10, jnp.float32),
                X[:, 0:1], Y[:, 0:1], Z[:, 0:1],
                jnp.zeros((_B, M), jnp.float32),
                jnp.zeros((_B, M), jnp.float32),
                jnp.zeros((_B, M), jnp.float32))
        carry = lax.fori_loop(0, M, step, init)
        ax, ay, az = carry[4], carry[5], carry[6]
        out_refs[3 * lvl][...] = ax
        out_refs[3 * lvl + 1][...] = ay
        out_refs[3 * lvl + 2][...] = az
        X, Y, Z = ax, ay, az


@functools.cache
def _fps_call():
    outs = []
    for (M, _, _) in _SA_CFG:
        outs += [jax.ShapeDtypeStruct((_B, M), jnp.float32)] * 3
    return pl.pallas_call(_fps_body, out_shape=tuple(outs))


def _ballq_body(q_ref, xt_ref, out_ref, *, r2, nsample, Nl):
    q = q_ref[0]
    xt = xt_ref[0]
    qq = jnp.sum(q * q, axis=1, keepdims=True)
    xx = jnp.sum(xt * xt, axis=0, keepdims=True)
    d = qq + xx - 2.0 * jnp.dot(q, xt, preferred_element_type=jnp.float32)
    d = jnp.maximum(d, 0.0)
    R = d.shape[0]
    iota = lax.broadcasted_iota(jnp.int32, (R, Nl), 1).astype(jnp.float32)
    cand = jnp.where(d <= r2, iota, float(Nl))
    cols = []
    for _ in range(nsample):
        m = jnp.min(cand, axis=1, keepdims=True)
        cols.append(m)
        cand = jnp.where(cand == m, float(Nl), cand)
    first = cols[0]
    for k in range(nsample):
        v = jnp.where(cols[k] >= float(Nl), first, cols[k])
        out_ref[0, :, pl.ds(k, 1)] = v.astype(jnp.int32)


@functools.cache
def _ballq_call(M, Nl, r2, nsample):
    R = min(128, M)
    body = functools.partial(_ballq_body, r2=r2, nsample=nsample, Nl=Nl)
    return pl.pallas_call(
        body,
        grid=(_B, M // R),
        in_specs=[
            pl.BlockSpec((1, R, 8), lambda b, i: (b, i, 0)),
            pl.BlockSpec((1, 8, Nl), lambda b, i: (b, 0, 0)),
        ],
        out_specs=pl.BlockSpec((1, R, nsample), lambda b, i: (b, i, 0)),
        out_shape=jax.ShapeDtypeStruct((_B, M, nsample), jnp.int32),
    )


@functools.cache
def _sc_gather_call(V, D, Rtot):
    assert Rtot % _SC_NW == 0
    rows_per_w = Rtot // _SC_NW
    assert rows_per_w % 8 == 0
    chunk = rows_per_w
    while chunk * D * 4 > 262144:
        chunk //= 2
    assert rows_per_w % chunk == 0 and chunk % 8 == 0
    n_chunks = rows_per_w // chunk
    mesh = plsc.VectorSubcoreMesh(core_axis_name="c", subcore_axis_name="s")

    @functools.partial(
        pl.kernel, mesh=mesh,
        compiler_params=pltpu.CompilerParams(use_tc_tiling_on_sc=False),
        out_type=jax.ShapeDtypeStruct((Rtot, D), jnp.float32),
        scratch_types=[
            pltpu.VMEM((chunk,), jnp.int32),
            pltpu.VMEM((chunk, D), jnp.float32),
            pltpu.SemaphoreType.DMA,
        ],
    )
    def k(table_hbm, idx_hbm, out_hbm, idx_v, rows_v, sem):
        wid = lax.axis_index("s") * _SC_NC + lax.axis_index("c")
        base = wid * rows_per_w
        for t in range(n_chunks):
            off = base + t * chunk
            pltpu.sync_copy(idx_hbm.at[pl.ds(off, chunk)], idx_v)
            pltpu.async_copy(table_hbm.at[idx_v], rows_v, sem).wait()
            pltpu.sync_copy(rows_v, out_hbm.at[pl.ds(off, chunk)])

    return k


def _gather(table, idx, Rtot, D):
    return _sc_gather_call(table.shape[0], D, Rtot)(table, idx)


def _sa_mlp_body(g_ref, c_ref, w1_ref, w2_ref, w3_ref, out_ref, *, nsample):
    g = g_ref[0]
    c = c_ref[0]
    Q, ns, Dp = g.shape
    g = g - c[:, None, :]
    gf = g.reshape(Q * ns, Dp)
    h = jax.nn.relu(jnp.dot(gf, w1_ref[...],
                            preferred_element_type=jnp.float32) * _BN_SCALE)
    h = jax.nn.relu(jnp.dot(h, w2_ref[...],
                            preferred_element_type=jnp.float32) * _BN_SCALE)
    h = jax.nn.relu(jnp.dot(h, w3_ref[...],
                            preferred_element_type=jnp.float32) * _BN_SCALE)
    h3 = h.reshape(Q, ns, h.shape[1])
    out_ref[0] = jnp.max(h3, axis=1)


@functools.cache
def _sa_mlp_call(M, Dp, C1, C2, C3, nsample):
    Q = min(128, M)
    body = functools.partial(_sa_mlp_body, nsample=nsample)
    return pl.pallas_call(
        body,
        grid=(_B, M // Q),
        in_specs=[
            pl.BlockSpec((1, Q, nsample, Dp), lambda b, i: (b, i, 0, 0)),
            pl.BlockSpec((1, Q, Dp), lambda b, i: (b, i, 0)),
            pl.BlockSpec((Dp, C1), lambda b, i: (0, 0)),
            pl.BlockSpec((C1, C2), lambda b, i: (0, 0)),
            pl.BlockSpec((C2, C3), lambda b, i: (0, 0)),
        ],
        out_specs=pl.BlockSpec((1, Q, C3), lambda b, i: (b, i, 0)),
        out_shape=jax.ShapeDtypeStruct((_B, M, C3), jnp.float32),
    )


def _fp_plan_body(q_ref, xt_ref, idx_ref, w_ref, *, Ns):
    q = q_ref[0]
    xt = xt_ref[0]
    qq = jnp.sum(q * q, axis=1, keepdims=True)
    xx = jnp.sum(xt * xt, axis=0, keepdims=True)
    d = qq + xx - 2.0 * jnp.dot(q, xt, preferred_element_type=jnp.float32)
    d = jnp.maximum(d, 0.0)
    R = d.shape[0]
    iota = lax.broadcasted_iota(jnp.int32, (R, Ns), 1).astype(jnp.float32)
    idxs, dists = [], []
    for _ in range(3):
        m = jnp.min(d, axis=1, keepdims=True)
        i = jnp.min(jnp.where(d == m, iota, float(Ns)), axis=1, keepdims=True)
        idxs.append(i)
        dists.append(m)
        d = jnp.where(iota == i, jnp.inf, d)
    recips = [1.0 / (dv + 1e-8) for dv in dists]
    tot = recips[0] + recips[1] + recips[2]
    for k in range(3):
        idx_ref[0, :, pl.ds(k, 1)] = idxs[k].astype(jnp.int32)
        w_ref[0, :, pl.ds(k, 1)] = recips[k] / tot


@functools.cache
def _fp_plan_call(Nl, Ns):
    R = min(128, Nl)
    body = functools.partial(_fp_plan_body, Ns=Ns)
    return pl.pallas_call(
        body,
        grid=(_B, Nl // R),
        in_specs=[
            pl.BlockSpec((1, R, 8), lambda b, i: (b, i, 0)),
            pl.BlockSpec((1, 8, Ns), lambda b, i: (b, 0, 0)),
        ],
        out_specs=[
            pl.BlockSpec((1, R, 3), lambda b, i: (b, i, 0)),
            pl.BlockSpec((1, R, 3), lambda b, i: (b, i, 0)),
        ],
        out_shape=[
            jax.ShapeDtypeStruct((_B, Nl, 3), jnp.int32),
            jax.ShapeDtypeStruct((_B, Nl, 3), jnp.float32),
        ],
    )


def _fp_mlp_body(g_ref, w_ref, s_ref, *rest, n_layers):
    wrefs = rest[:-1]
    out_ref = rest[-1]
    g = g_ref[0]
    wv = w_ref[0]
    skip = s_ref[0]
    interp = (g[:, 0, :] * wv[:, 0:1] + g[:, 1, :] * wv[:, 1:2]
              + g[:, 2, :] * wv[:, 2:3])
    h = jnp.dot(interp, wrefs[0][...], preferred_element_type=jnp.float32)
    h = h + jnp.dot(skip, wrefs[1][...], preferred_element_type=jnp.float32)
    h = jax.nn.relu(h * _BN_SCALE)
    for j in range(n_layers - 1):
        h = jax.nn.relu(jnp.dot(h, wrefs[2 + j][...],
                                preferred_element_type=jnp.float32) * _BN_SCALE)
    out_ref[0] = h


@functools.cache
def _fp_mlp_call(Nl, C, Csp, layer_dims):
    Q = min(128, Nl)
    n_layers = len(layer_dims)
    C1 = layer_dims[0][1]
    Cout = layer_dims[-1][1]
    in_specs = [
        pl.BlockSpec((1, Q, 3, C), lambda b, i: (b, i, 0, 0)),
        pl.BlockSpec((1, Q, 3), lambda b, i: (b, i, 0)),
        pl.BlockSpec((1, Q, Csp), lambda b, i: (b, i, 0)),
        pl.BlockSpec((C, C1), lambda b, i: (0, 0)),
        pl.BlockSpec((Csp, C1), lambda b, i: (0, 0)),
    ]
    for j in range(1, n_layers):
        ci, co = layer_dims[j]
        in_specs.append(pl.BlockSpec((ci, co), lambda b, i: (0, 0)))
    body = functools.partial(_fp_mlp_body, n_layers=n_layers)
    return pl.pallas_call(
        body,
        grid=(_B, Nl // Q),
        in_specs=in_specs,
        out_specs=pl.BlockSpec((1, Q, Cout), lambda b, i: (b, i, 0)),
        out_shape=jax.ShapeDtypeStruct((_B, Nl, Cout), jnp.float32),
    )


def _pad_cols(x, D):
    pad = D - x.shape[-1]
    if pad == 0:
        return x
    return jnp.concatenate(
        [x, jnp.zeros(x.shape[:-1] + (pad,), x.dtype)], axis=-1)


def _qpad(xyz):
    return _pad_cols(xyz, 8)


def _xyzT(xyz):
    return jnp.transpose(_pad_cols(xyz, 8), (0, 2, 1))


def _flat_idx(idx, Nl):
    b = jnp.arange(_B, dtype=jnp.int32).reshape((_B,) + (1,) * (idx.ndim - 1))
    return (idx + b * Nl).reshape(-1)


def _round16(c):
    return ((c + 15) // 16) * 16


def kernel(pointcloud, params):
    B, N, _ = pointcloud.shape
    xyz = pointcloud[..., 0:3]

    fps_outs = _fps_call()(xyz[..., 0], xyz[..., 1], xyz[..., 2])
    l_xyz = [xyz]
    for lvl in range(4):
        l_xyz.append(jnp.stack(fps_outs[3 * lvl:3 * lvl + 3], axis=-1))

    l_feats = [pointcloud]
    for i, (M, radius, nsample) in enumerate(_SA_CFG):
        cx, cf = l_xyz[i], l_feats[i]
        Nl = cx.shape[1]
        new_xyz = l_xyz[i + 1]
        gidx = _ballq_call(M, Nl, radius * radius, nsample)(
            _qpad(new_xyz), _xyzT(cx))
        Dp = _round16(3 + cf.shape[-1])
        table = _pad_cols(jnp.concatenate([cx, cf], axis=-1), Dp)
        rows = _gather(table.reshape(B * Nl, Dp), _flat_idx(gidx, Nl),
                       B * M * nsample, Dp)
        rows = rows.reshape(B, M, nsample, Dp)
        cen = _pad_cols(new_xyz, Dp)
        W1, W2, W3 = params['sa'][i]
        w1 = _pad_cols(W1, Dp).T
        w2, w3 = W2.T, W3.T
        feats = _sa_mlp_call(M, Dp, w1.shape[1], w2.shape[1], w3.shape[1],
                             nsample)(rows, cen, w1, w2, w3)
        l_feats.append(feats)

    for i in range(-1, -5, -1):
        big_xyz, small_xyz = l_xyz[i - 1], l_xyz[i]
        Nl, Ns = big_xyz.shape[1], small_xyz.shape[1]
        idx, w = _fp_plan_call(Nl, Ns)(_qpad(big_xyz), _xyzT(small_xyz))
        feats = l_feats[i]
        C = feats.shape[-1]
        rows = _gather(feats.reshape(B * Ns, C), _flat_idx(idx, Ns),
                       B * Nl * 3, C)
        rows = rows.reshape(B, Nl, 3, C)
        skip = l_feats[i - 1]
        Csp = _round16(skip.shape[-1])
        skip_p = _pad_cols(skip, Csp)
        Ws = params['fp'][i]
        W1 = Ws[0]
        w1a = W1[:, :C].T
        w1b = _pad_cols(W1[:, C:], Csp).T
        wrest = [Wj.T for Wj in Ws[1:]]
        layer_dims = tuple((Wj.shape[1], Wj.shape[0]) for Wj in Ws)
        l_feats[i - 1] = _fp_mlp_call(Nl, C, Csp, layer_dims)(
            rows, w, skip_p, w1a, w1b, *wrest)

    return jnp.transpose(l_feats[0], (0, 2, 1))


# speedup vs baseline: 20.1904x; 1.1374x over previous
import functools

import jax
import jax.numpy as jnp
from jax import lax
from jax.experimental import pallas as pl
from jax.experimental.pallas import tpu as pltpu
from jax.experimental.pallas import tpu_sc as plsc
import numpy as np

_BN_SCALE = 1.0 / np.sqrt(1.0 + 1e-5)
_SA_CFG = [(2048, 0.1, 32), (512, 0.2, 32), (128, 0.4, 32), (32, 0.8, 32)]
_B = 4
_N = 8192
_SC_NC = 2
_SC_NS = 16
_SC_NW = _SC_NC * _SC_NS


_FPS_F = 8


def _flat_iota(shape):
    i = lax.broadcasted_iota(jnp.int32, shape, 1)
    j = lax.broadcasted_iota(jnp.int32, shape, 2)
    return (i * shape[2] + j).astype(jnp.float32)


def _fps_body(x_ref, y_ref, z_ref, *out_refs):
    X = x_ref[...]
    Y = y_ref[...]
    Z = z_ref[...]
    npoints = [c[0] for c in _SA_CFG]
    for lvl, M in enumerate(npoints):
        Nl = X.shape[1] * X.shape[2]
        Wm = M // _FPS_F
        iota_n = _flat_iota(X.shape)
        iota_m = _flat_iota((_B, _FPS_F, Wm))

        def step(s, carry, X=X, Y=Y, Z=Z,
                 iota_n=iota_n, iota_m=iota_m, Nl=Nl):
            dists, cx, cy, cz, ax, ay, az = carry
            sel = iota_m == s.astype(jnp.float32)
            ax = jnp.where(sel, cx, ax)
            ay = jnp.where(sel, cy, ay)
            az = jnp.where(sel, cz, az)
            d = (X - cx) ** 2 + (Y - cy) ** 2 + (Z - cz) ** 2
            dists = jnp.minimum(dists, d)
            m = jnp.max(jnp.max(dists, axis=2, keepdims=True),
                        axis=1, keepdims=True)
            cand = jnp.where(dists == m, iota_n, float(Nl))
            far = jnp.min(jnp.min(cand, axis=2, keepdims=True),
                          axis=1, keepdims=True)
            fmask = iota_n == far
            cx = jnp.sum(jnp.sum(jnp.where(fmask, X, 0.0), axis=2,
                                 keepdims=True), axis=1, keepdims=True)
            cy = jnp.sum(jnp.sum(jnp.where(fmask, Y, 0.0), axis=2,
                                 keepdims=True), axis=1, keepdims=True)
            cz = jnp.sum(jnp.sum(jnp.where(fmask, Z, 0.0), axis=2,
                                 keepdims=True), axis=1, keepdims=True)
            return (dists, cx, cy, cz, ax, ay, az)

        init = (jnp.full(X.shape, 1e10, jnp.float32),
                X[:, 0:1, 0:1], Y[:, 0:1, 0:1], Z[:, 0:1, 0:1],
                jnp.zeros((_B, _FPS_F, Wm), jnp.float32),
                jnp.zeros((_B, _FPS_F, Wm), jnp.float32),
                jnp.zeros((_B, _FPS_F, Wm), jnp.float32))
        carry = lax.fori_loop(0, M, step, init)
        ax, ay, az = carry[4], carry[5], carry[6]
        out_refs[3 * lvl][...] = ax
        out_refs[3 * lvl + 1][...] = ay
        out_refs[3 * lvl + 2][...] = az
        X, Y, Z = ax, ay, az


@functools.cache
def _fps_call():
    outs = []
    for (M, _, _) in _SA_CFG:
        outs += [jax.ShapeDtypeStruct((_B, _FPS_F, M // _FPS_F),
                                      jnp.float32)] * 3
    return pl.pallas_call(_fps_body, out_shape=tuple(outs))


def _ballq_body(q_ref, xt_ref, out_ref, *, r2, nsample, Nl):
    q = q_ref[0]
    xt = xt_ref[0]
    qq = jnp.sum(q * q, axis=1, keepdims=True)
    xx = jnp.sum(xt * xt, axis=0, keepdims=True)
    d = qq + xx - 2.0 * jnp.dot(q, xt, preferred_element_type=jnp.float32)
    d = jnp.maximum(d, 0.0)
    R = d.shape[0]
    iota = lax.broadcasted_iota(jnp.int32, (R, Nl), 1).astype(jnp.float32)
    cand = jnp.where(d <= r2, iota, float(Nl))
    cols = []
    for _ in range(nsample):
        m = jnp.min(cand, axis=1, keepdims=True)
        cols.append(m)
        cand = jnp.where(cand == m, float(Nl), cand)
    first = cols[0]
    for k in range(nsample):
        v = jnp.where(cols[k] >= float(Nl), first, cols[k])
        out_ref[0, :, pl.ds(k, 1)] = v.astype(jnp.int32)


@functools.cache
def _ballq_call(M, Nl, r2, nsample):
    R = min(128, M)
    body = functools.partial(_ballq_body, r2=r2, nsample=nsample, Nl=Nl)
    return pl.pallas_call(
        body,
        grid=(_B, M // R),
        in_specs=[
            pl.BlockSpec((1, R, 8), lambda b, i: (b, i, 0)),
            pl.BlockSpec((1, 8, Nl), lambda b, i: (b, 0, 0)),
        ],
        out_specs=pl.BlockSpec((1, R, nsample), lambda b, i: (b, i, 0)),
        out_shape=jax.ShapeDtypeStruct((_B, M, nsample), jnp.int32),
    )


@functools.cache
def _sc_gather_call(V, D, Rtot):
    assert Rtot % _SC_NW == 0
    rows_per_w = Rtot // _SC_NW
    assert rows_per_w % 8 == 0
    chunk = rows_per_w
    while chunk * D * 4 > 262144:
        chunk //= 2
    assert rows_per_w % chunk == 0 and chunk % 8 == 0
    n_chunks = rows_per_w // chunk
    mesh = plsc.VectorSubcoreMesh(core_axis_name="c", subcore_axis_name="s")

    @functools.partial(
        pl.kernel, mesh=mesh,
        compiler_params=pltpu.CompilerParams(use_tc_tiling_on_sc=False),
        out_type=jax.ShapeDtypeStruct((Rtot, D), jnp.float32),
        scratch_types=[
            pltpu.VMEM((chunk,), jnp.int32),
            pltpu.VMEM((chunk, D), jnp.float32),
            pltpu.SemaphoreType.DMA,
        ],
    )
    def k(table_hbm, idx_hbm, out_hbm, idx_v, rows_v, sem):
        wid = lax.axis_index("s") * _SC_NC + lax.axis_index("c")
        base = wid * rows_per_w
        for t in range(n_chunks):
            off = base + t * chunk
            pltpu.sync_copy(idx_hbm.at[pl.ds(off, chunk)], idx_v)
            pltpu.async_copy(table_hbm.at[idx_v], rows_v, sem).wait()
            pltpu.sync_copy(rows_v, out_hbm.at[pl.ds(off, chunk)])

    return k


def _gather(table, idx, Rtot, D):
    return _sc_gather_call(table.shape[0], D, Rtot)(table, idx)


def _sa_mlp_body(g_ref, c_ref, w1_ref, w2_ref, w3_ref, out_ref, *, nsample):
    g = g_ref[0]
    c = c_ref[0]
    Q, ns, Dp = g.shape
    g = g - c[:, None, :]
    gf = g.reshape(Q * ns, Dp)
    h = jax.nn.relu(jnp.dot(gf, w1_ref[...],
                            preferred_element_type=jnp.float32) * _BN_SCALE)
    h = jax.nn.relu(jnp.dot(h, w2_ref[...],
                            preferred_element_type=jnp.float32) * _BN_SCALE)
    h = jax.nn.relu(jnp.dot(h, w3_ref[...],
                            preferred_element_type=jnp.float32) * _BN_SCALE)
    h3 = h.reshape(Q, ns, h.shape[1])
    out_ref[0] = jnp.max(h3, axis=1)


@functools.cache
def _sa_mlp_call(M, Dp, C1, C2, C3, nsample):
    Q = min(128, M)
    body = functools.partial(_sa_mlp_body, nsample=nsample)
    return pl.pallas_call(
        body,
        grid=(_B, M // Q),
        in_specs=[
            pl.BlockSpec((1, Q, nsample, Dp), lambda b, i: (b, i, 0, 0)),
            pl.BlockSpec((1, Q, Dp), lambda b, i: (b, i, 0)),
            pl.BlockSpec((Dp, C1), lambda b, i: (0, 0)),
            pl.BlockSpec((C1, C2), lambda b, i: (0, 0)),
            pl.BlockSpec((C2, C3), lambda b, i: (0, 0)),
        ],
        out_specs=pl.BlockSpec((1, Q, C3), lambda b, i: (b, i, 0)),
        out_shape=jax.ShapeDtypeStruct((_B, M, C3), jnp.float32),
    )


def _fp_plan_body(q_ref, xt_ref, idx_ref, w_ref, *, Ns):
    q = q_ref[0]
    xt = xt_ref[0]
    qq = jnp.sum(q * q, axis=1, keepdims=True)
    xx = jnp.sum(xt * xt, axis=0, keepdims=True)
    d = qq + xx - 2.0 * jnp.dot(q, xt, preferred_element_type=jnp.float32)
    d = jnp.maximum(d, 0.0)
    R = d.shape[0]
    iota = lax.broadcasted_iota(jnp.int32, (R, Ns), 1).astype(jnp.float32)
    idxs, dists = [], []
    for _ in range(3):
        m = jnp.min(d, axis=1, keepdims=True)
        i = jnp.min(jnp.where(d == m, iota, float(Ns)), axis=1, keepdims=True)
        idxs.append(i)
        dists.append(m)
        d = jnp.where(iota == i, jnp.inf, d)
    recips = [1.0 / (dv + 1e-8) for dv in dists]
    tot = recips[0] + recips[1] + recips[2]
    for k in range(3):
        idx_ref[0, :, pl.ds(k, 1)] = idxs[k].astype(jnp.int32)
        w_ref[0, :, pl.ds(k, 1)] = recips[k] / tot


@functools.cache
def _fp_plan_call(Nl, Ns):
    R = min(128, Nl)
    body = functools.partial(_fp_plan_body, Ns=Ns)
    return pl.pallas_call(
        body,
        grid=(_B, Nl // R),
        in_specs=[
            pl.BlockSpec((1, R, 8), lambda b, i: (b, i, 0)),
            pl.BlockSpec((1, 8, Ns), lambda b, i: (b, 0, 0)),
        ],
        out_specs=[
            pl.BlockSpec((1, R, 3), lambda b, i: (b, i, 0)),
            pl.BlockSpec((1, R, 3), lambda b, i: (b, i, 0)),
        ],
        out_shape=[
            jax.ShapeDtypeStruct((_B, Nl, 3), jnp.int32),
            jax.ShapeDtypeStruct((_B, Nl, 3), jnp.float32),
        ],
    )


def _fp_mlp_body(g_ref, w_ref, s_ref, *rest, n_layers):
    wrefs = rest[:-1]
    out_ref = rest[-1]
    g = g_ref[0]
    wv = w_ref[0]
    skip = s_ref[0]
    interp = (g[:, 0, :] * wv[:, 0:1] + g[:, 1, :] * wv[:, 1:2]
              + g[:, 2, :] * wv[:, 2:3])
    h = jnp.dot(interp, wrefs[0][...], preferred_element_type=jnp.float32)
    h = h + jnp.dot(skip, wrefs[1][...], preferred_element_type=jnp.float32)
    h = jax.nn.relu(h * _BN_SCALE)
    for j in range(n_layers - 1):
        h = jax.nn.relu(jnp.dot(h, wrefs[2 + j][...],
                                preferred_element_type=jnp.float32) * _BN_SCALE)
    out_ref[0] = h


@functools.cache
def _fp_mlp_call(Nl, C, Csp, layer_dims):
    Q = min(128, Nl)
    n_layers = len(layer_dims)
    C1 = layer_dims[0][1]
    Cout = layer_dims[-1][1]
    in_specs = [
        pl.BlockSpec((1, Q, 3, C), lambda b, i: (b, i, 0, 0)),
        pl.BlockSpec((1, Q, 3), lambda b, i: (b, i, 0)),
        pl.BlockSpec((1, Q, Csp), lambda b, i: (b, i, 0)),
        pl.BlockSpec((C, C1), lambda b, i: (0, 0)),
        pl.BlockSpec((Csp, C1), lambda b, i: (0, 0)),
    ]
    for j in range(1, n_layers):
        ci, co = layer_dims[j]
        in_specs.append(pl.BlockSpec((ci, co), lambda b, i: (0, 0)))
    body = functools.partial(_fp_mlp_body, n_layers=n_layers)
    return pl.pallas_call(
        body,
        grid=(_B, Nl // Q),
        in_specs=in_specs,
        out_specs=pl.BlockSpec((1, Q, Cout), lambda b, i: (b, i, 0)),
        out_shape=jax.ShapeDtypeStruct((_B, Nl, Cout), jnp.float32),
    )


def _pad_cols(x, D):
    pad = D - x.shape[-1]
    if pad == 0:
        return x
    return jnp.concatenate(
        [x, jnp.zeros(x.shape[:-1] + (pad,), x.dtype)], axis=-1)


def _qpad(xyz):
    return _pad_cols(xyz, 8)


def _xyzT(xyz):
    return jnp.transpose(_pad_cols(xyz, 8), (0, 2, 1))


def _flat_idx(idx, Nl):
    b = jnp.arange(_B, dtype=jnp.int32).reshape((_B,) + (1,) * (idx.ndim - 1))
    return (idx + b * Nl).reshape(-1)


def _round16(c):
    return ((c + 15) // 16) * 16


def kernel(pointcloud, params):
    B, N, _ = pointcloud.shape
    xyz = pointcloud[..., 0:3]

    fps_outs = _fps_call()(
        xyz[..., 0].reshape(B, _FPS_F, N // _FPS_F),
        xyz[..., 1].reshape(B, _FPS_F, N // _FPS_F),
        xyz[..., 2].reshape(B, _FPS_F, N // _FPS_F))
    l_xyz = [xyz]
    for lvl in range(4):
        M = _SA_CFG[lvl][0]
        l_xyz.append(jnp.stack(
            [o.reshape(B, M) for o in fps_outs[3 * lvl:3 * lvl + 3]],
            axis=-1))

    l_feats = [pointcloud]
    for i, (M, radius, nsample) in enumerate(_SA_CFG):
        cx, cf = l_xyz[i], l_feats[i]
        Nl = cx.shape[1]
        new_xyz = l_xyz[i + 1]
        gidx = _ballq_call(M, Nl, radius * radius, nsample)(
            _qpad(new_xyz), _xyzT(cx))
        Dp = _round16(3 + cf.shape[-1])
        table = _pad_cols(jnp.concatenate([cx, cf], axis=-1), Dp)
        rows = _gather(table.reshape(B * Nl, Dp), _flat_idx(gidx, Nl),
                       B * M * nsample, Dp)
        rows = rows.reshape(B, M, nsample, Dp)
        cen = _pad_cols(new_xyz, Dp)
        W1, W2, W3 = params['sa'][i]
        w1 = _pad_cols(W1, Dp).T
        w2, w3 = W2.T, W3.T
        feats = _sa_mlp_call(M, Dp, w1.shape[1], w2.shape[1], w3.shape[1],
                             nsample)(rows, cen, w1, w2, w3)
        l_feats.append(feats)

    for i in range(-1, -5, -1):
        big_xyz, small_xyz = l_xyz[i - 1], l_xyz[i]
        Nl, Ns = big_xyz.shape[1], small_xyz.shape[1]
        idx, w = _fp_plan_call(Nl, Ns)(_qpad(big_xyz), _xyzT(small_xyz))
        feats = l_feats[i]
        C = feats.shape[-1]
        rows = _gather(feats.reshape(B * Ns, C), _flat_idx(idx, Ns),
                       B * Nl * 3, C)
        rows = rows.reshape(B, Nl, 3, C)
        skip = l_feats[i - 1]
        Csp = _round16(skip.shape[-1])
        skip_p = _pad_cols(skip, Csp)
        Ws = params['fp'][i]
        W1 = Ws[0]
        w1a = W1[:, :C].T
        w1b = _pad_cols(W1[:, C:], Csp).T
        wrest = [Wj.T for Wj in Ws[1:]]
        layer_dims = tuple((Wj.shape[1], Wj.shape[0]) for Wj in Ws)
        l_feats[i - 1] = _fp_mlp_call(Nl, C, Csp, layer_dims)(
            rows, w, skip_p, w1a, w1b, *wrest)

    return jnp.transpose(l_feats[0], (0, 2, 1))


# speedup vs baseline: 24.8234x; 1.2295x over previous
import functools

import jax
import jax.numpy as jnp
from jax import lax
from jax.experimental import pallas as pl
from jax.experimental.pallas import tpu as pltpu
from jax.experimental.pallas import tpu_sc as plsc
import numpy as np

_BN_SCALE = 1.0 / np.sqrt(1.0 + 1e-5)
_SA_CFG = [(2048, 0.1, 32), (512, 0.2, 32), (128, 0.4, 32), (32, 0.8, 32)]
_B = 4
_N = 8192
_SC_NC = 2
_SC_NS = 16
_SC_NW = _SC_NC * _SC_NS


_FPS_F = 8


def _flat_iota(shape):
    i = lax.broadcasted_iota(jnp.int32, shape, 1)
    j = lax.broadcasted_iota(jnp.int32, shape, 2)
    return (i * shape[2] + j).astype(jnp.float32)


def _fps_body(x_ref, y_ref, z_ref, *out_refs):
    X = x_ref[...]
    Y = y_ref[...]
    Z = z_ref[...]
    npoints = [c[0] for c in _SA_CFG]
    for lvl, M in enumerate(npoints):
        Nl = X.shape[1] * X.shape[2]
        Wm = M // _FPS_F
        iota_n = _flat_iota(X.shape)
        iota_m = _flat_iota((_B, _FPS_F, Wm))

        def step(s, carry, X=X, Y=Y, Z=Z,
                 iota_n=iota_n, iota_m=iota_m, Nl=Nl):
            dists, cx, cy, cz, ax, ay, az = carry
            sel = iota_m == s.astype(jnp.float32)
            ax = jnp.where(sel, cx, ax)
            ay = jnp.where(sel, cy, ay)
            az = jnp.where(sel, cz, az)
            d = (X - cx) ** 2 + (Y - cy) ** 2 + (Z - cz) ** 2
            dists = jnp.minimum(dists, d)
            m = jnp.max(jnp.max(dists, axis=2, keepdims=True),
                        axis=1, keepdims=True)
            cand = jnp.where(dists == m, iota_n, float(Nl))
            far = jnp.min(jnp.min(cand, axis=2, keepdims=True),
                          axis=1, keepdims=True)
            fmask = iota_n == far
            cx = jnp.sum(jnp.sum(jnp.where(fmask, X, 0.0), axis=2,
                                 keepdims=True), axis=1, keepdims=True)
            cy = jnp.sum(jnp.sum(jnp.where(fmask, Y, 0.0), axis=2,
                                 keepdims=True), axis=1, keepdims=True)
            cz = jnp.sum(jnp.sum(jnp.where(fmask, Z, 0.0), axis=2,
                                 keepdims=True), axis=1, keepdims=True)
            return (dists, cx, cy, cz, ax, ay, az)

        init = (jnp.full(X.shape, 1e10, jnp.float32),
                X[:, 0:1, 0:1], Y[:, 0:1, 0:1], Z[:, 0:1, 0:1],
                jnp.zeros((_B, _FPS_F, Wm), jnp.float32),
                jnp.zeros((_B, _FPS_F, Wm), jnp.float32),
                jnp.zeros((_B, _FPS_F, Wm), jnp.float32))
        carry = lax.fori_loop(0, M, step, init)
        ax, ay, az = carry[4], carry[5], carry[6]
        out_refs[3 * lvl][...] = ax
        out_refs[3 * lvl + 1][...] = ay
        out_refs[3 * lvl + 2][...] = az
        X, Y, Z = ax, ay, az


@functools.cache
def _fps_call():
    outs = []
    for (M, _, _) in _SA_CFG:
        outs += [jax.ShapeDtypeStruct((_B, _FPS_F, M // _FPS_F),
                                      jnp.float32)] * 3
    return pl.pallas_call(_fps_body, out_shape=tuple(outs))


def _ballq_body(q_ref, xt_ref, out_ref, *, r2, nsample, Nl):
    q = q_ref[0]
    xt = xt_ref[0]
    qq = jnp.sum(q * q, axis=1, keepdims=True)
    xx = jnp.sum(xt * xt, axis=0, keepdims=True)
    d = qq + xx - 2.0 * jnp.dot(q, xt, preferred_element_type=jnp.float32)
    d = jnp.maximum(d, 0.0)
    R = d.shape[0]
    L = Nl // 16
    bits = jnp.zeros((R, L), jnp.int32)
    for w in range(16):
        mask = d[:, w * L:(w + 1) * L] <= r2
        bits = bits + (mask.astype(jnp.int32) << w)
    iota_l = lax.broadcasted_iota(jnp.int32, (R, L), 1)
    cols = []
    for _ in range(nsample):
        low = bits & (-bits)
        f = low.astype(jnp.float32)
        e = (lax.bitcast_convert_type(f, jnp.int32) >> 23) - 127
        jc = jnp.where(low > 0, (e * L + iota_l).astype(jnp.float32),
                       float(Nl))
        m = jnp.min(jc, axis=1, keepdims=True)
        cols.append(m)
        bits = jnp.where(jc == m, bits - low, bits)
    first = cols[0]
    for k in range(nsample):
        v = jnp.where(cols[k] >= float(Nl), first, cols[k])
        out_ref[0, :, pl.ds(k, 1)] = v.astype(jnp.int32)


@functools.cache
def _ballq_call(M, Nl, r2, nsample):
    R = min(128, M)
    body = functools.partial(_ballq_body, r2=r2, nsample=nsample, Nl=Nl)
    return pl.pallas_call(
        body,
        grid=(_B, M // R),
        in_specs=[
            pl.BlockSpec((1, R, 8), lambda b, i: (b, i, 0)),
            pl.BlockSpec((1, 8, Nl), lambda b, i: (b, 0, 0)),
        ],
        out_specs=pl.BlockSpec((1, R, nsample), lambda b, i: (b, i, 0)),
        out_shape=jax.ShapeDtypeStruct((_B, M, nsample), jnp.int32),
    )


@functools.cache
def _sc_gather_call(V, D, Rtot):
    assert Rtot % _SC_NW == 0
    rows_per_w = Rtot // _SC_NW
    assert rows_per_w % 8 == 0
    chunk = rows_per_w
    while chunk * D * 4 > 262144:
        chunk //= 2
    assert rows_per_w % chunk == 0 and chunk % 8 == 0
    n_chunks = rows_per_w // chunk
    mesh = plsc.VectorSubcoreMesh(core_axis_name="c", subcore_axis_name="s")

    @functools.partial(
        pl.kernel, mesh=mesh,
        compiler_params=pltpu.CompilerParams(use_tc_tiling_on_sc=False),
        out_type=jax.ShapeDtypeStruct((Rtot, D), jnp.float32),
        scratch_types=[
            pltpu.VMEM((chunk,), jnp.int32),
            pltpu.VMEM((chunk, D), jnp.float32),
            pltpu.SemaphoreType.DMA,
        ],
    )
    def k(table_hbm, idx_hbm, out_hbm, idx_v, rows_v, sem):
        wid = lax.axis_index("s") * _SC_NC + lax.axis_index("c")
        base = wid * rows_per_w
        for t in range(n_chunks):
            off = base + t * chunk
            pltpu.sync_copy(idx_hbm.at[pl.ds(off, chunk)], idx_v)
            pltpu.async_copy(table_hbm.at[idx_v], rows_v, sem).wait()
            pltpu.sync_copy(rows_v, out_hbm.at[pl.ds(off, chunk)])

    return k


def _gather(table, idx, Rtot, D):
    return _sc_gather_call(table.shape[0], D, Rtot)(table, idx)


def _sa_mlp_body(g_ref, c_ref, w1_ref, w2_ref, w3_ref, out_ref, *, nsample):
    g = g_ref[0]
    c = c_ref[0]
    Q, ns, Dp = g.shape
    g = g - c[:, None, :]
    gf = g.reshape(Q * ns, Dp)
    h = jax.nn.relu(jnp.dot(gf, w1_ref[...],
                            preferred_element_type=jnp.float32) * _BN_SCALE)
    h = jax.nn.relu(jnp.dot(h, w2_ref[...],
                            preferred_element_type=jnp.float32) * _BN_SCALE)
    h = jax.nn.relu(jnp.dot(h, w3_ref[...],
                            preferred_element_type=jnp.float32) * _BN_SCALE)
    h3 = h.reshape(Q, ns, h.shape[1])
    out_ref[0] = jnp.max(h3, axis=1)


@functools.cache
def _sa_mlp_call(M, Dp, C1, C2, C3, nsample):
    Q = min(128, M)
    body = functools.partial(_sa_mlp_body, nsample=nsample)
    return pl.pallas_call(
        body,
        grid=(_B, M // Q),
        in_specs=[
            pl.BlockSpec((1, Q, nsample, Dp), lambda b, i: (b, i, 0, 0)),
            pl.BlockSpec((1, Q, Dp), lambda b, i: (b, i, 0)),
            pl.BlockSpec((Dp, C1), lambda b, i: (0, 0)),
            pl.BlockSpec((C1, C2), lambda b, i: (0, 0)),
            pl.BlockSpec((C2, C3), lambda b, i: (0, 0)),
        ],
        out_specs=pl.BlockSpec((1, Q, C3), lambda b, i: (b, i, 0)),
        out_shape=jax.ShapeDtypeStruct((_B, M, C3), jnp.float32),
    )


def _fp_plan_body(q_ref, xt_ref, idx_ref, w_ref, *, Ns):
    q = q_ref[0]
    xt = xt_ref[0]
    qq = jnp.sum(q * q, axis=1, keepdims=True)
    xx = jnp.sum(xt * xt, axis=0, keepdims=True)
    d = qq + xx - 2.0 * jnp.dot(q, xt, preferred_element_type=jnp.float32)
    d = jnp.maximum(d, 0.0)
    R = d.shape[0]
    iota = lax.broadcasted_iota(jnp.int32, (R, Ns), 1).astype(jnp.float32)
    idxs, dists = [], []
    for _ in range(3):
        m = jnp.min(d, axis=1, keepdims=True)
        i = jnp.min(jnp.where(d == m, iota, float(Ns)), axis=1, keepdims=True)
        idxs.append(i)
        dists.append(m)
        d = jnp.where(iota == i, jnp.inf, d)
    recips = [1.0 / (dv + 1e-8) for dv in dists]
    tot = recips[0] + recips[1] + recips[2]
    for k in range(3):
        idx_ref[0, :, pl.ds(k, 1)] = idxs[k].astype(jnp.int32)
        w_ref[0, :, pl.ds(k, 1)] = recips[k] / tot


@functools.cache
def _fp_plan_call(Nl, Ns):
    R = min(128, Nl)
    body = functools.partial(_fp_plan_body, Ns=Ns)
    return pl.pallas_call(
        body,
        grid=(_B, Nl // R),
        in_specs=[
            pl.BlockSpec((1, R, 8), lambda b, i: (b, i, 0)),
            pl.BlockSpec((1, 8, Ns), lambda b, i: (b, 0, 0)),
        ],
        out_specs=[
            pl.BlockSpec((1, R, 3), lambda b, i: (b, i, 0)),
            pl.BlockSpec((1, R, 3), lambda b, i: (b, i, 0)),
        ],
        out_shape=[
            jax.ShapeDtypeStruct((_B, Nl, 3), jnp.int32),
            jax.ShapeDtypeStruct((_B, Nl, 3), jnp.float32),
        ],
    )


def _fp_mlp_body(g_ref, w_ref, s_ref, *rest, n_layers):
    wrefs = rest[:-1]
    out_ref = rest[-1]
    g = g_ref[0]
    wv = w_ref[0]
    skip = s_ref[0]
    interp = (g[:, 0, :] * wv[:, 0:1] + g[:, 1, :] * wv[:, 1:2]
              + g[:, 2, :] * wv[:, 2:3])
    h = jnp.dot(interp, wrefs[0][...], preferred_element_type=jnp.float32)
    h = h + jnp.dot(skip, wrefs[1][...], preferred_element_type=jnp.float32)
    h = jax.nn.relu(h * _BN_SCALE)
    for j in range(n_layers - 1):
        h = jax.nn.relu(jnp.dot(h, wrefs[2 + j][...],
                                preferred_element_type=jnp.float32) * _BN_SCALE)
    out_ref[0] = h


@functools.cache
def _fp_mlp_call(Nl, C, Csp, layer_dims):
    Q = min(128, Nl)
    n_layers = len(layer_dims)
    C1 = layer_dims[0][1]
    Cout = layer_dims[-1][1]
    in_specs = [
        pl.BlockSpec((1, Q, 3, C), lambda b, i: (b, i, 0, 0)),
        pl.BlockSpec((1, Q, 3), lambda b, i: (b, i, 0)),
        pl.BlockSpec((1, Q, Csp), lambda b, i: (b, i, 0)),
        pl.BlockSpec((C, C1), lambda b, i: (0, 0)),
        pl.BlockSpec((Csp, C1), lambda b, i: (0, 0)),
    ]
    for j in range(1, n_layers):
        ci, co = layer_dims[j]
        in_specs.append(pl.BlockSpec((ci, co), lambda b, i: (0, 0)))
    body = functools.partial(_fp_mlp_body, n_layers=n_layers)
    return pl.pallas_call(
        body,
        grid=(_B, Nl // Q),
        in_specs=in_specs,
        out_specs=pl.BlockSpec((1, Q, Cout), lambda b, i: (b, i, 0)),
        out_shape=jax.ShapeDtypeStruct((_B, Nl, Cout), jnp.float32),
    )


def _pad_cols(x, D):
    pad = D - x.shape[-1]
    if pad == 0:
        return x
    return jnp.concatenate(
        [x, jnp.zeros(x.shape[:-1] + (pad,), x.dtype)], axis=-1)


def _qpad(xyz):
    return _pad_cols(xyz, 8)


def _xyzT(xyz):
    return jnp.transpose(_pad_cols(xyz, 8), (0, 2, 1))


def _flat_idx(idx, Nl):
    b = jnp.arange(_B, dtype=jnp.int32).reshape((_B,) + (1,) * (idx.ndim - 1))
    return (idx + b * Nl).reshape(-1)


def _round16(c):
    return ((c + 15) // 16) * 16


def kernel(pointcloud, params):
    B, N, _ = pointcloud.shape
    xyz = pointcloud[..., 0:3]

    fps_outs = _fps_call()(
        xyz[..., 0].reshape(B, _FPS_F, N // _FPS_F),
        xyz[..., 1].reshape(B, _FPS_F, N // _FPS_F),
        xyz[..., 2].reshape(B, _FPS_F, N // _FPS_F))
    l_xyz = [xyz]
    for lvl in range(4):
        M = _SA_CFG[lvl][0]
        l_xyz.append(jnp.stack(
            [o.reshape(B, M) for o in fps_outs[3 * lvl:3 * lvl + 3]],
            axis=-1))

    l_feats = [pointcloud]
    for i, (M, radius, nsample) in enumerate(_SA_CFG):
        cx, cf = l_xyz[i], l_feats[i]
        Nl = cx.shape[1]
        new_xyz = l_xyz[i + 1]
        gidx = _ballq_call(M, Nl, radius * radius, nsample)(
            _qpad(new_xyz), _xyzT(cx))
        Dp = _round16(3 + cf.shape[-1])
        table = _pad_cols(jnp.concatenate([cx, cf], axis=-1), Dp)
        rows = _gather(table.reshape(B * Nl, Dp), _flat_idx(gidx, Nl),
                       B * M * nsample, Dp)
        rows = rows.reshape(B, M, nsample, Dp)
        cen = _pad_cols(new_xyz, Dp)
        W1, W2, W3 = params['sa'][i]
        w1 = _pad_cols(W1, Dp).T
        w2, w3 = W2.T, W3.T
        feats = _sa_mlp_call(M, Dp, w1.shape[1], w2.shape[1], w3.shape[1],
                             nsample)(rows, cen, w1, w2, w3)
        l_feats.append(feats)

    for i in range(-1, -5, -1):
        big_xyz, small_xyz = l_xyz[i - 1], l_xyz[i]
        Nl, Ns = big_xyz.shape[1], small_xyz.shape[1]
        idx, w = _fp_plan_call(Nl, Ns)(_qpad(big_xyz), _xyzT(small_xyz))
        feats = l_feats[i]
        C = feats.shape[-1]
        rows = _gather(feats.reshape(B * Ns, C), _flat_idx(idx, Ns),
                       B * Nl * 3, C)
        rows = rows.reshape(B, Nl, 3, C)
        skip = l_feats[i - 1]
        Csp = _round16(skip.shape[-1])
        skip_p = _pad_cols(skip, Csp)
        Ws = params['fp'][i]
        W1 = Ws[0]
        w1a = W1[:, :C].T
        w1b = _pad_cols(W1[:, C:], Csp).T
        wrest = [Wj.T for Wj in Ws[1:]]
        layer_dims = tuple((Wj.shape[1], Wj.shape[0]) for Wj in Ws)
        l_feats[i - 1] = _fp_mlp_call(Nl, C, Csp, layer_dims)(
            rows, w, skip_p, w1a, w1b, *wrest)

    return jnp.transpose(l_feats[0], (0, 2, 1))


# speedup vs baseline: 24.8365x; 1.0005x over previous
import functools

import jax
import jax.numpy as jnp
from jax import lax
from jax.experimental import pallas as pl
from jax.experimental.pallas import tpu as pltpu
from jax.experimental.pallas import tpu_sc as plsc
import numpy as np

_BN_SCALE = 1.0 / np.sqrt(1.0 + 1e-5)
_SA_CFG = [(2048, 0.1, 32), (512, 0.2, 32), (128, 0.4, 32), (32, 0.8, 32)]
_B = 4
_N = 8192
_SC_NC = 2
_SC_NS = 16
_SC_NW = _SC_NC * _SC_NS


_FPS_F = 8


def _flat_iota(shape):
    i = lax.broadcasted_iota(jnp.int32, shape, 1)
    j = lax.broadcasted_iota(jnp.int32, shape, 2)
    return (i * shape[2] + j).astype(jnp.float32)


def _fps_body(x_ref, y_ref, z_ref, *out_refs):
    X = x_ref[...]
    Y = y_ref[...]
    Z = z_ref[...]
    npoints = [c[0] for c in _SA_CFG]
    for lvl, M in enumerate(npoints):
        Nl = X.shape[1] * X.shape[2]
        Wm = M // _FPS_F
        iota_n = _flat_iota(X.shape)
        iota_m = _flat_iota((_B, _FPS_F, Wm))

        def step(s, carry, X=X, Y=Y, Z=Z,
                 iota_n=iota_n, iota_m=iota_m, Nl=Nl):
            dists, cx, cy, cz, ax, ay, az = carry
            sel = iota_m == s.astype(jnp.float32)
            ax = jnp.where(sel, cx, ax)
            ay = jnp.where(sel, cy, ay)
            az = jnp.where(sel, cz, az)
            d = (X - cx) ** 2 + (Y - cy) ** 2 + (Z - cz) ** 2
            dists = jnp.minimum(dists, d)
            m = jnp.max(jnp.max(dists, axis=2, keepdims=True),
                        axis=1, keepdims=True)
            cand = jnp.where(dists == m, iota_n, float(Nl))
            far = jnp.min(jnp.min(cand, axis=2, keepdims=True),
                          axis=1, keepdims=True)
            fmask = iota_n == far
            cx = jnp.sum(jnp.sum(jnp.where(fmask, X, 0.0), axis=2,
                                 keepdims=True), axis=1, keepdims=True)
            cy = jnp.sum(jnp.sum(jnp.where(fmask, Y, 0.0), axis=2,
                                 keepdims=True), axis=1, keepdims=True)
            cz = jnp.sum(jnp.sum(jnp.where(fmask, Z, 0.0), axis=2,
                                 keepdims=True), axis=1, keepdims=True)
            return (dists, cx, cy, cz, ax, ay, az)

        init = (jnp.full(X.shape, 1e10, jnp.float32),
                X[:, 0:1, 0:1], Y[:, 0:1, 0:1], Z[:, 0:1, 0:1],
                jnp.zeros((_B, _FPS_F, Wm), jnp.float32),
                jnp.zeros((_B, _FPS_F, Wm), jnp.float32),
                jnp.zeros((_B, _FPS_F, Wm), jnp.float32))
        carry = lax.fori_loop(0, M, step, init)
        ax, ay, az = carry[4], carry[5], carry[6]
        out_refs[3 * lvl][...] = ax
        out_refs[3 * lvl + 1][...] = ay
        out_refs[3 * lvl + 2][...] = az
        X, Y, Z = ax, ay, az


@functools.cache
def _fps_call():
    outs = []
    for (M, _, _) in _SA_CFG:
        outs += [jax.ShapeDtypeStruct((_B, _FPS_F, M // _FPS_F),
                                      jnp.float32)] * 3
    return pl.pallas_call(_fps_body, out_shape=tuple(outs))


def _ballq_body(q_ref, xt_ref, out_ref, *, r2, nsample, Nl):
    q = q_ref[0]
    xt = xt_ref[0]
    qq = jnp.sum(q * q, axis=1, keepdims=True)
    xx = jnp.sum(xt * xt, axis=0, keepdims=True)
    d = qq + xx - 2.0 * jnp.dot(q, xt, preferred_element_type=jnp.float32)
    d = jnp.maximum(d, 0.0)
    R = d.shape[0]
    L = Nl // 16
    bits = jnp.zeros((R, L), jnp.int32)
    for w in range(16):
        mask = d[:, w * L:(w + 1) * L] <= r2
        bits = bits + (mask.astype(jnp.int32) << w)
    iota_l = lax.broadcasted_iota(jnp.int32, (R, L), 1)
    cols = []
    for _ in range(nsample):
        low = bits & (-bits)
        f = low.astype(jnp.float32)
        e = (lax.bitcast_convert_type(f, jnp.int32) >> 23) - 127
        jc = jnp.where(low > 0, (e * L + iota_l).astype(jnp.float32),
                       float(Nl))
        m = jnp.min(jc, axis=1, keepdims=True)
        cols.append(m)
        bits = jnp.where(jc == m, bits - low, bits)
    first = cols[0]
    for k in range(nsample):
        v = jnp.where(cols[k] >= float(Nl), first, cols[k])
        out_ref[0, :, pl.ds(k, 1)] = v.astype(jnp.int32)


@functools.cache
def _ballq_call(M, Nl, r2, nsample):
    R = min(128, M)
    body = functools.partial(_ballq_body, r2=r2, nsample=nsample, Nl=Nl)
    return pl.pallas_call(
        body,
        grid=(_B, M // R),
        in_specs=[
            pl.BlockSpec((1, R, 8), lambda b, i: (b, i, 0)),
            pl.BlockSpec((1, 8, Nl), lambda b, i: (b, 0, 0)),
        ],
        out_specs=pl.BlockSpec((1, R, nsample), lambda b, i: (b, i, 0)),
        out_shape=jax.ShapeDtypeStruct((_B, M, nsample), jnp.int32),
    )


@functools.cache
def _sc_gather_call(V, D, Rtot):
    assert Rtot % _SC_NW == 0
    rows_per_w = Rtot // _SC_NW
    assert rows_per_w % 8 == 0
    chunk = rows_per_w
    while chunk * D * 4 > 131072:
        chunk //= 2
    assert rows_per_w % chunk == 0 and chunk % 8 == 0
    n_chunks = rows_per_w // chunk
    mesh = plsc.VectorSubcoreMesh(core_axis_name="c", subcore_axis_name="s")

    @functools.partial(
        pl.kernel, mesh=mesh,
        compiler_params=pltpu.CompilerParams(use_tc_tiling_on_sc=False),
        out_type=jax.ShapeDtypeStruct((Rtot, D), jnp.float32),
        scratch_types=[
            pltpu.VMEM((chunk,), jnp.int32),
            pltpu.VMEM((chunk,), jnp.int32),
            pltpu.VMEM((chunk, D), jnp.float32),
            pltpu.VMEM((chunk, D), jnp.float32),
            pltpu.SemaphoreType.DMA,
            pltpu.SemaphoreType.DMA,
        ],
    )
    def k(table_hbm, idx_hbm, out_hbm, idx_v0, idx_v1, rows_v0, rows_v1,
          sem0, sem1):
        wid = lax.axis_index("s") * _SC_NC + lax.axis_index("c")
        base = wid * rows_per_w
        bufs = ((idx_v0, rows_v0, sem0), (idx_v1, rows_v1, sem1))
        pltpu.sync_copy(idx_hbm.at[pl.ds(base, chunk)], idx_v0)
        handles = [pltpu.async_copy(table_hbm.at[idx_v0], rows_v0, sem0)]
        for t in range(n_chunks):
            _, rv, _ = bufs[t % 2]
            if t + 1 < n_chunks:
                niv, nrv, nsm = bufs[(t + 1) % 2]
                noff = base + (t + 1) * chunk
                pltpu.sync_copy(idx_hbm.at[pl.ds(noff, chunk)], niv)
                handles.append(pltpu.async_copy(table_hbm.at[niv], nrv, nsm))
            handles[t].wait()
            pltpu.sync_copy(rv, out_hbm.at[pl.ds(base + t * chunk, chunk)])

    return k


def _gather(table, idx, Rtot, D):
    return _sc_gather_call(table.shape[0], D, Rtot)(table, idx)


def _sa_mlp_body(g_ref, c_ref, w1_ref, w2_ref, w3_ref, out_ref, *, nsample):
    g = g_ref[0]
    c = c_ref[0]
    Q, ns, Dp = g.shape
    g = g - c[:, None, :]
    gf = g.reshape(Q * ns, Dp)
    h = jax.nn.relu(jnp.dot(gf, w1_ref[...],
                            preferred_element_type=jnp.float32) * _BN_SCALE)
    h = jax.nn.relu(jnp.dot(h, w2_ref[...],
                            preferred_element_type=jnp.float32) * _BN_SCALE)
    h = jax.nn.relu(jnp.dot(h, w3_ref[...],
                            preferred_element_type=jnp.float32) * _BN_SCALE)
    h3 = h.reshape(Q, ns, h.shape[1])
    out_ref[0] = jnp.max(h3, axis=1)


@functools.cache
def _sa_mlp_call(M, Dp, C1, C2, C3, nsample):
    Q = min(128, M)
    body = functools.partial(_sa_mlp_body, nsample=nsample)
    return pl.pallas_call(
        body,
        grid=(_B, M // Q),
        in_specs=[
            pl.BlockSpec((1, Q, nsample, Dp), lambda b, i: (b, i, 0, 0)),
            pl.BlockSpec((1, Q, Dp), lambda b, i: (b, i, 0)),
            pl.BlockSpec((Dp, C1), lambda b, i: (0, 0)),
            pl.BlockSpec((C1, C2), lambda b, i: (0, 0)),
            pl.BlockSpec((C2, C3), lambda b, i: (0, 0)),
        ],
        out_specs=pl.BlockSpec((1, Q, C3), lambda b, i: (b, i, 0)),
        out_shape=jax.ShapeDtypeStruct((_B, M, C3), jnp.float32),
    )


def _fp_plan_body(q_ref, xt_ref, idx_ref, w_ref, *, Ns):
    q = q_ref[0]
    xt = xt_ref[0]
    qq = jnp.sum(q * q, axis=1, keepdims=True)
    xx = jnp.sum(xt * xt, axis=0, keepdims=True)
    d = qq + xx - 2.0 * jnp.dot(q, xt, preferred_element_type=jnp.float32)
    d = jnp.maximum(d, 0.0)
    R = d.shape[0]
    iota = lax.broadcasted_iota(jnp.int32, (R, Ns), 1).astype(jnp.float32)
    idxs, dists = [], []
    for _ in range(3):
        m = jnp.min(d, axis=1, keepdims=True)
        i = jnp.min(jnp.where(d == m, iota, float(Ns)), axis=1, keepdims=True)
        idxs.append(i)
        dists.append(m)
        d = jnp.where(iota == i, jnp.inf, d)
    recips = [1.0 / (dv + 1e-8) for dv in dists]
    tot = recips[0] + recips[1] + recips[2]
    for k in range(3):
        idx_ref[0, :, pl.ds(k, 1)] = idxs[k].astype(jnp.int32)
        w_ref[0, :, pl.ds(k, 1)] = recips[k] / tot


@functools.cache
def _fp_plan_call(Nl, Ns):
    R = min(128, Nl)
    body = functools.partial(_fp_plan_body, Ns=Ns)
    return pl.pallas_call(
        body,
        grid=(_B, Nl // R),
        in_specs=[
            pl.BlockSpec((1, R, 8), lambda b, i: (b, i, 0)),
            pl.BlockSpec((1, 8, Ns), lambda b, i: (b, 0, 0)),
        ],
        out_specs=[
            pl.BlockSpec((1, R, 3), lambda b, i: (b, i, 0)),
            pl.BlockSpec((1, R, 3), lambda b, i: (b, i, 0)),
        ],
        out_shape=[
            jax.ShapeDtypeStruct((_B, Nl, 3), jnp.int32),
            jax.ShapeDtypeStruct((_B, Nl, 3), jnp.float32),
        ],
    )


def _fp_mlp_body(g_ref, w_ref, s_ref, *rest, n_layers):
    wrefs = rest[:-1]
    out_ref = rest[-1]
    g = g_ref[0]
    wv = w_ref[0]
    skip = s_ref[0]
    interp = (g[:, 0, :] * wv[:, 0:1] + g[:, 1, :] * wv[:, 1:2]
              + g[:, 2, :] * wv[:, 2:3])
    h = jnp.dot(interp, wrefs[0][...], preferred_element_type=jnp.float32)
    h = h + jnp.dot(skip, wrefs[1][...], preferred_element_type=jnp.float32)
    h = jax.nn.relu(h * _BN_SCALE)
    for j in range(n_layers - 1):
        h = jax.nn.relu(jnp.dot(h, wrefs[2 + j][...],
                                preferred_element_type=jnp.float32) * _BN_SCALE)
    out_ref[0] = h


@functools.cache
def _fp_mlp_call(Nl, C, Csp, layer_dims):
    Q = min(128, Nl)
    n_layers = len(layer_dims)
    C1 = layer_dims[0][1]
    Cout = layer_dims[-1][1]
    in_specs = [
        pl.BlockSpec((1, Q, 3, C), lambda b, i: (b, i, 0, 0)),
        pl.BlockSpec((1, Q, 3), lambda b, i: (b, i, 0)),
        pl.BlockSpec((1, Q, Csp), lambda b, i: (b, i, 0)),
        pl.BlockSpec((C, C1), lambda b, i: (0, 0)),
        pl.BlockSpec((Csp, C1), lambda b, i: (0, 0)),
    ]
    for j in range(1, n_layers):
        ci, co = layer_dims[j]
        in_specs.append(pl.BlockSpec((ci, co), lambda b, i: (0, 0)))
    body = functools.partial(_fp_mlp_body, n_layers=n_layers)
    return pl.pallas_call(
        body,
        grid=(_B, Nl // Q),
        in_specs=in_specs,
        out_specs=pl.BlockSpec((1, Q, Cout), lambda b, i: (b, i, 0)),
        out_shape=jax.ShapeDtypeStruct((_B, Nl, Cout), jnp.float32),
    )


def _pad_cols(x, D):
    pad = D - x.shape[-1]
    if pad == 0:
        return x
    return jnp.concatenate(
        [x, jnp.zeros(x.shape[:-1] + (pad,), x.dtype)], axis=-1)


def _qpad(xyz):
    return _pad_cols(xyz, 8)


def _xyzT(xyz):
    return jnp.transpose(_pad_cols(xyz, 8), (0, 2, 1))


def _flat_idx(idx, Nl):
    b = jnp.arange(_B, dtype=jnp.int32).reshape((_B,) + (1,) * (idx.ndim - 1))
    return (idx + b * Nl).reshape(-1)


def _round16(c):
    return ((c + 15) // 16) * 16


def kernel(pointcloud, params):
    B, N, _ = pointcloud.shape
    xyz = pointcloud[..., 0:3]

    fps_outs = _fps_call()(
        xyz[..., 0].reshape(B, _FPS_F, N // _FPS_F),
        xyz[..., 1].reshape(B, _FPS_F, N // _FPS_F),
        xyz[..., 2].reshape(B, _FPS_F, N // _FPS_F))
    l_xyz = [xyz]
    for lvl in range(4):
        M = _SA_CFG[lvl][0]
        l_xyz.append(jnp.stack(
            [o.reshape(B, M) for o in fps_outs[3 * lvl:3 * lvl + 3]],
            axis=-1))

    l_feats = [pointcloud]
    for i, (M, radius, nsample) in enumerate(_SA_CFG):
        cx, cf = l_xyz[i], l_feats[i]
        Nl = cx.shape[1]
        new_xyz = l_xyz[i + 1]
        gidx = _ballq_call(M, Nl, radius * radius, nsample)(
            _qpad(new_xyz), _xyzT(cx))
        Dp = _round16(3 + cf.shape[-1])
        table = _pad_cols(jnp.concatenate([cx, cf], axis=-1), Dp)
        rows = _gather(table.reshape(B * Nl, Dp), _flat_idx(gidx, Nl),
                       B * M * nsample, Dp)
        rows = rows.reshape(B, M, nsample, Dp)
        cen = _pad_cols(new_xyz, Dp)
        W1, W2, W3 = params['sa'][i]
        w1 = _pad_cols(W1, Dp).T
        w2, w3 = W2.T, W3.T
        feats = _sa_mlp_call(M, Dp, w1.shape[1], w2.shape[1], w3.shape[1],
                             nsample)(rows, cen, w1, w2, w3)
        l_feats.append(feats)

    for i in range(-1, -5, -1):
        big_xyz, small_xyz = l_xyz[i - 1], l_xyz[i]
        Nl, Ns = big_xyz.shape[1], small_xyz.shape[1]
        idx, w = _fp_plan_call(Nl, Ns)(_qpad(big_xyz), _xyzT(small_xyz))
        feats = l_feats[i]
        C = feats.shape[-1]
        rows = _gather(feats.reshape(B * Ns, C), _flat_idx(idx, Ns),
                       B * Nl * 3, C)
        rows = rows.reshape(B, Nl, 3, C)
        skip = l_feats[i - 1]
        Csp = _round16(skip.shape[-1])
        skip_p = _pad_cols(skip, Csp)
        Ws = params['fp'][i]
        W1 = Ws[0]
        w1a = W1[:, :C].T
        w1b = _pad_cols(W1[:, C:], Csp).T
        wrest = [Wj.T for Wj in Ws[1:]]
        layer_dims = tuple((Wj.shape[1], Wj.shape[0]) for Wj in Ws)
        l_feats[i - 1] = _fp_mlp_call(Nl, C, Csp, layer_dims)(
            rows, w, skip_p, w1a, w1b, *wrest)

    return jnp.transpose(l_feats[0], (0, 2, 1))
